# baseline (device time: 52820 ns/iter reference)
import jax
import jax.numpy as jnp
from jax import lax
from jax.experimental import pallas as pl
from jax.experimental.pallas import tpu as pltpu

N_DEV = 4
B = 2
S_SHARD = 256
HQ = 4
DH = 64
HD = HQ * DH
S_FULL = N_DEV * S_SHARD
NEG = -1e9

_sem_signal = getattr(pl, "semaphore_signal", None) or pltpu.semaphore_signal
_sem_wait = getattr(pl, "semaphore_wait", None) or pltpu.semaphore_wait
_DeviceIdType = getattr(pl, "DeviceIdType", None) or pltpu.DeviceIdType


def _body(x_ref, wq_ref, kv_ref, wo_ref, out_ref,
          kv_full, local_sem, send_sems, recv_sems):
    my_pos = lax.axis_index("i")
    right = lax.rem(my_pos + 1, N_DEV)
    left = lax.rem(my_pos + N_DEV - 1, N_DEV)

    barrier_sem = pltpu.get_barrier_semaphore()
    for nbr in (left, right):
        _sem_signal(barrier_sem, inc=1, device_id=(nbr,),
                    device_id_type=_DeviceIdType.MESH)
    _sem_wait(barrier_sem, 2)

    cp = pltpu.make_async_copy(kv_ref, kv_full.at[my_pos], local_sem)
    cp.start()
    cp.wait()

    for h in range(N_DEV - 1):
        send_origin = lax.rem(my_pos - h + N_DEV, N_DEV)
        rdma = pltpu.make_async_remote_copy(
            src_ref=kv_full.at[send_origin],
            dst_ref=kv_full.at[send_origin],
            send_sem=send_sems.at[h],
            recv_sem=recv_sems.at[h],
            device_id=(right,),
            device_id_type=_DeviceIdType.MESH,
        )
        rdma.start()
        rdma.wait()

    q0 = my_pos * S_SHARD
    qi = q0 + lax.broadcasted_iota(jnp.int32, (S_SHARD, S_FULL), 0)
    ki = lax.broadcasted_iota(jnp.int32, (S_SHARD, S_FULL), 1)
    mask = (jnp.abs(qi - ki) <= 128) | (ki < 32) | (qi < 32)

    dn = (((1,), (1,)), ((), ()))
    for b in range(B):
        q_b = jnp.dot(x_ref[b], wq_ref[...],
                      preferred_element_type=jnp.float32)
        kv_b = [kv_full[o, b] for o in range(N_DEV)]
        ctx_parts = []
        for h in range(HQ):
            q_bh = q_b[:, h * DH:(h + 1) * DH]
            s = jnp.concatenate(
                [lax.dot_general(q_bh, kv_b[o][:, h * DH:(h + 1) * DH], dn,
                                 preferred_element_type=jnp.float32)
                 for o in range(N_DEV)], axis=1) * 0.125
            s = jnp.where(mask, s, NEG)
            m = jnp.max(s, axis=1, keepdims=True)
            w = jnp.exp(s - m)
            w = w / jnp.sum(w, axis=1, keepdims=True)
            acc = jnp.zeros((S_SHARD, DH), jnp.float32)
            for o in range(N_DEV):
                acc = acc + jnp.dot(
                    w[:, o * S_SHARD:(o + 1) * S_SHARD],
                    kv_b[o][:, HD + h * DH:HD + (h + 1) * DH],
                    preferred_element_type=jnp.float32)
            ctx_parts.append(acc)
        ctx_b = jnp.concatenate(ctx_parts, axis=1)
        out_ref[b] = jnp.dot(ctx_b, wo_ref[...],
                             preferred_element_type=jnp.float32)


def kernel(x, Wq, K_ext, V_ext, Wo):
    b, s, hq, dh = K_ext.shape
    kv = jnp.concatenate(
        [K_ext.reshape(b, s, hq * dh), V_ext.reshape(b, s, hq * dh)],
        axis=-1)
    return pl.pallas_call(
        _body,
        out_shape=jax.ShapeDtypeStruct(x.shape, jnp.float32),
        in_specs=[pl.BlockSpec(memory_space=pltpu.VMEM)] * 4,
        out_specs=pl.BlockSpec(memory_space=pltpu.VMEM),
        scratch_shapes=[
            pltpu.VMEM((N_DEV, B, S_SHARD, 2 * HD), jnp.float32),
            pltpu.SemaphoreType.DMA,
            pltpu.SemaphoreType.DMA((N_DEV - 1,)),
            pltpu.SemaphoreType.DMA((N_DEV - 1,)),
        ],
        compiler_params=pltpu.CompilerParams(collective_id=0),
    )(x, Wq, kv, Wo)


# device time: 30608 ns/iter; 1.7257x vs baseline; 1.7257x over previous
import jax
import jax.numpy as jnp
from jax import lax
from jax.experimental import pallas as pl
from jax.experimental.pallas import tpu as pltpu

N_DEV = 4
B = 2
S = 256
HQ = 4
DH = 64
HD = HQ * DH
KVW = 2 * B * HD
NEG = -1e9
HALF = S // 2

_sem_signal = getattr(pl, "semaphore_signal", None) or pltpu.semaphore_signal
_sem_wait = getattr(pl, "semaphore_wait", None) or pltpu.semaphore_wait
_DeviceIdType = getattr(pl, "DeviceIdType", None) or pltpu.DeviceIdType


def _block_mask(qi, off):
    ki = off + lax.broadcasted_iota(jnp.int32, (S, S), 1)
    return (jnp.abs(qi - ki) <= 128) | (ki < 32) | (qi < 32)


def _body(x_ref, wq_ref, kv_ref, wo_ref, out_ref, kv_full, ss, rs):
    my_pos = lax.axis_index("i")
    right = lax.rem(my_pos + 1, N_DEV)
    left = lax.rem(my_pos + N_DEV - 1, N_DEV)

    barrier_sem = pltpu.get_barrier_semaphore()
    for nbr in (left, right):
        _sem_signal(barrier_sem, inc=1, device_id=(nbr,),
                    device_id_type=_DeviceIdType.MESH)
    _sem_wait(barrier_sem, 2)

    dA_left = pltpu.make_async_remote_copy(
        src_ref=kv_ref, dst_ref=kv_full.at[0],
        send_sem=ss.at[0], recv_sem=rs.at[0],
        device_id=(left,), device_id_type=_DeviceIdType.MESH)
    dA_right = pltpu.make_async_remote_copy(
        src_ref=kv_ref, dst_ref=kv_full.at[2],
        send_sem=ss.at[1], recv_sem=rs.at[1],
        device_id=(right,), device_id_type=_DeviceIdType.MESH)
    dA_left.start()
    dA_right.start()

    q = [jnp.dot(x_ref[b], wq_ref[...],
                 preferred_element_type=jnp.float32) * 0.125
         for b in range(B)]

    dA_left.wait_recv()
    dA_right.wait_recv()

    dB_left = pltpu.make_async_remote_copy(
        src_ref=kv_full.at[0, pl.ds(0, HALF)],
        dst_ref=kv_full.at[1, pl.ds(0, HALF)],
        send_sem=ss.at[2], recv_sem=rs.at[2],
        device_id=(left,), device_id_type=_DeviceIdType.MESH)
    dB_right = pltpu.make_async_remote_copy(
        src_ref=kv_full.at[2, pl.ds(HALF, HALF)],
        dst_ref=kv_full.at[1, pl.ds(HALF, HALF)],
        send_sem=ss.at[3], recv_sem=rs.at[3],
        device_id=(right,), device_id_type=_DeviceIdType.MESH)
    dB_left.start()
    dB_right.start()

    qi = my_pos * S + lax.broadcasted_iota(jnp.int32, (S, S), 0)
    off_own = my_pos * S
    off_r0 = lax.rem(my_pos + 1, N_DEV) * S
    off_far = lax.rem(my_pos + 2, N_DEV) * S
    off_r2 = lax.rem(my_pos + 3, N_DEV) * S
    mask1 = jnp.concatenate(
        [_block_mask(qi, off_own), _block_mask(qi, off_r0),
         _block_mask(qi, off_r2)], axis=1)
    mask_far = _block_mask(qi, off_far)

    dn = (((1,), (1,)), ((), ()))

    def group1(b, h):
        q_bh = q[b][:, h * DH:(h + 1) * DH]
        c0 = b * 2 * HD + h * DH
        cv = b * 2 * HD + HD + h * DH
        blocks = [kv_ref[...], kv_full[0], kv_full[2]]
        s1 = jnp.concatenate(
            [lax.dot_general(q_bh, blk[:, c0:c0 + DH], dn,
                             preferred_element_type=jnp.float32)
             for blk in blocks], axis=1)
        s1 = jnp.where(mask1, s1, NEG)
        m1 = jnp.max(s1, axis=1, keepdims=True)
        w1 = jnp.exp(s1 - m1)
        l1 = jnp.sum(w1, axis=1, keepdims=True)
        acc = jnp.zeros((S, DH), jnp.float32)
        for j, blk in enumerate(blocks):
            acc = acc + jnp.dot(w1[:, j * S:(j + 1) * S],
                                blk[:, cv:cv + DH],
                                preferred_element_type=jnp.float32)
        return m1, l1, acc

    part = [[group1(b, h) for h in range(HQ)] for b in range(B)]

    dB_left.wait_recv()
    dB_right.wait_recv()

    for b in range(B):
        ctx_parts = []
        for h in range(HQ):
            m1, l1, acc = part[b][h]
            q_bh = q[b][:, h * DH:(h + 1) * DH]
            c0 = b * 2 * HD + h * DH
            cv = b * 2 * HD + HD + h * DH
            s2 = lax.dot_general(q_bh, kv_full[1][:, c0:c0 + DH], dn,
                                 preferred_element_type=jnp.float32)
            s2 = jnp.where(mask_far, s2, NEG)
            m2 = jnp.maximum(m1, jnp.max(s2, axis=1, keepdims=True))
            alpha = jnp.exp(m1 - m2)
            w2 = jnp.exp(s2 - m2)
            l = l1 * alpha + jnp.sum(w2, axis=1, keepdims=True)
            acc = acc * alpha + jnp.dot(w2, kv_full[1][:, cv:cv + DH],
                                        preferred_element_type=jnp.float32)
            ctx_parts.append(acc / l)
        ctx_b = jnp.concatenate(ctx_parts, axis=1)
        out_ref[b] = jnp.dot(ctx_b, wo_ref[...],
                             preferred_element_type=jnp.float32)

    dA_left.wait_send()
    dA_right.wait_send()
    dB_left.wait_send()
    dB_right.wait_send()


def kernel(x, Wq, K_ext, V_ext, Wo):
    b, s, hq, dh = K_ext.shape
    K2 = K_ext.reshape(b, s, hq * dh)
    V2 = V_ext.reshape(b, s, hq * dh)
    kv = jnp.concatenate([K2[0], V2[0], K2[1], V2[1]], axis=-1)
    return pl.pallas_call(
        _body,
        out_shape=jax.ShapeDtypeStruct(x.shape, jnp.float32),
        in_specs=[pl.BlockSpec(memory_space=pltpu.VMEM)] * 4,
        out_specs=pl.BlockSpec(memory_space=pltpu.VMEM),
        scratch_shapes=[
            pltpu.VMEM((3, S, KVW), jnp.float32),
            pltpu.SemaphoreType.DMA((4,)),
            pltpu.SemaphoreType.DMA((4,)),
        ],
        compiler_params=pltpu.CompilerParams(collective_id=0),
    )(x, Wq, kv, Wo)


# device time: 29907 ns/iter; 1.7661x vs baseline; 1.0234x over previous
import jax
import jax.numpy as jnp
from jax import lax
from jax.experimental import pallas as pl
from jax.experimental.pallas import tpu as pltpu

N_DEV = 4
B = 2
S = 256
HQ = 4
DH = 64
HD = HQ * DH
KVW = 2 * B * HD
NEG = -1e9
HALF = S // 2

_sem_signal = getattr(pl, "semaphore_signal", None) or pltpu.semaphore_signal
_sem_wait = getattr(pl, "semaphore_wait", None) or pltpu.semaphore_wait
_DeviceIdType = getattr(pl, "DeviceIdType", None) or pltpu.DeviceIdType


def _block_mask(qi, off):
    ki = off + lax.broadcasted_iota(jnp.int32, (S, S), 1)
    return (jnp.abs(qi - ki) <= 128) | (ki < 32) | (qi < 32)


def _body(x_ref, wq_ref, k_ref, v_ref, wo_ref, out_ref, kv_s, kv_full,
          ss, rs):
    my_pos = lax.axis_index("i")
    right = lax.rem(my_pos + 1, N_DEV)
    left = lax.rem(my_pos + N_DEV - 1, N_DEV)

    for b in range(B):
        kv_s[:, b * 2 * HD:b * 2 * HD + HD] = k_ref[b].reshape(S, HD)
        kv_s[:, b * 2 * HD + HD:(b + 1) * 2 * HD] = v_ref[b].reshape(S, HD)

    barrier_sem = pltpu.get_barrier_semaphore()
    for nbr in (left, right):
        _sem_signal(barrier_sem, inc=1, device_id=(nbr,),
                    device_id_type=_DeviceIdType.MESH)
    _sem_wait(barrier_sem, 2)

    dA_left = pltpu.make_async_remote_copy(
        src_ref=kv_s, dst_ref=kv_full.at[0],
        send_sem=ss.at[0], recv_sem=rs.at[0],
        device_id=(left,), device_id_type=_DeviceIdType.MESH)
    dA_right = pltpu.make_async_remote_copy(
        src_ref=kv_s, dst_ref=kv_full.at[2],
        send_sem=ss.at[1], recv_sem=rs.at[1],
        device_id=(right,), device_id_type=_DeviceIdType.MESH)
    dA_left.start()
    dA_right.start()

    q = [jnp.dot(x_ref[b], wq_ref[...],
                 preferred_element_type=jnp.float32) * 0.125
         for b in range(B)]

    dA_left.wait_recv()
    dA_right.wait_recv()

    dB_left = pltpu.make_async_remote_copy(
        src_ref=kv_full.at[0, pl.ds(0, HALF)],
        dst_ref=kv_full.at[1, pl.ds(0, HALF)],
        send_sem=ss.at[2], recv_sem=rs.at[2],
        device_id=(left,), device_id_type=_DeviceIdType.MESH)
    dB_right = pltpu.make_async_remote_copy(
        src_ref=kv_full.at[2, pl.ds(HALF, HALF)],
        dst_ref=kv_full.at[1, pl.ds(HALF, HALF)],
        send_sem=ss.at[3], recv_sem=rs.at[3],
        device_id=(right,), device_id_type=_DeviceIdType.MESH)
    dB_left.start()
    dB_right.start()

    qi = my_pos * S + lax.broadcasted_iota(jnp.int32, (S, S), 0)
    off_own = my_pos * S
    off_r0 = lax.rem(my_pos + 1, N_DEV) * S
    off_far = lax.rem(my_pos + 2, N_DEV) * S
    off_r2 = lax.rem(my_pos + 3, N_DEV) * S
    mask1 = jnp.concatenate(
        [_block_mask(qi, off_own), _block_mask(qi, off_r0),
         _block_mask(qi, off_r2)], axis=1)
    mask_far = _block_mask(qi, off_far)

    dn = (((1,), (1,)), ((), ()))

    def group1(b, h):
        q_bh = q[b][:, h * DH:(h + 1) * DH]
        c0 = b * 2 * HD + h * DH
        cv = b * 2 * HD + HD + h * DH
        blocks = [kv_s[...], kv_full[0], kv_full[2]]
        s1 = jnp.concatenate(
            [lax.dot_general(q_bh, blk[:, c0:c0 + DH], dn,
                             preferred_element_type=jnp.float32)
             for blk in blocks], axis=1)
        s1 = jnp.where(mask1, s1, NEG)
        m1 = jnp.max(s1, axis=1, keepdims=True)
        w1 = jnp.exp(s1 - m1)
        l1 = jnp.sum(w1, axis=1, keepdims=True)
        acc = jnp.zeros((S, DH), jnp.float32)
        for j, blk in enumerate(blocks):
            acc = acc + jnp.dot(w1[:, j * S:(j + 1) * S],
                                blk[:, cv:cv + DH],
                                preferred_element_type=jnp.float32)
        return m1, l1, acc

    part = [[group1(b, h) for h in range(HQ)] for b in range(B)]

    dB_left.wait_recv()
    dB_right.wait_recv()

    for b in range(B):
        ctx_parts = []
        for h in range(HQ):
            m1, l1, acc = part[b][h]
            q_bh = q[b][:, h * DH:(h + 1) * DH]
            c0 = b * 2 * HD + h * DH
            cv = b * 2 * HD + HD + h * DH
            s2 = lax.dot_general(q_bh, kv_full[1][:, c0:c0 + DH], dn,
                                 preferred_element_type=jnp.float32)
            s2 = jnp.where(mask_far, s2, NEG)
            m2 = jnp.maximum(m1, jnp.max(s2, axis=1, keepdims=True))
            alpha = jnp.exp(m1 - m2)
            w2 = jnp.exp(s2 - m2)
            l = l1 * alpha + jnp.sum(w2, axis=1, keepdims=True)
            acc = acc * alpha + jnp.dot(w2, kv_full[1][:, cv:cv + DH],
                                        preferred_element_type=jnp.float32)
            ctx_parts.append(acc / l)
        ctx_b = jnp.concatenate(ctx_parts, axis=1)
        out_ref[b] = jnp.dot(ctx_b, wo_ref[...],
                             preferred_element_type=jnp.float32)

    dA_left.wait_send()
    dA_right.wait_send()
    dB_left.wait_send()
    dB_right.wait_send()


def kernel(x, Wq, K_ext, V_ext, Wo):
    return pl.pallas_call(
        _body,
        out_shape=jax.ShapeDtypeStruct(x.shape, jnp.float32),
        in_specs=[pl.BlockSpec(memory_space=pltpu.VMEM)] * 5,
        out_specs=pl.BlockSpec(memory_space=pltpu.VMEM),
        scratch_shapes=[
            pltpu.VMEM((S, KVW), jnp.float32),
            pltpu.VMEM((3, S, KVW), jnp.float32),
            pltpu.SemaphoreType.DMA((4,)),
            pltpu.SemaphoreType.DMA((4,)),
        ],
        compiler_params=pltpu.CompilerParams(collective_id=0),
    )(x, Wq, K_ext, V_ext, Wo)
